# baseline (speedup 1.0000x reference)
import functools

import jax
import jax.numpy as jnp
from jax import lax
from jax.experimental import pallas as pl
from jax.experimental.pallas import tpu as pltpu
from jax.experimental.pallas import tpu_sc as plsc

_NC = 2
_NS = 16
_NW = _NC * _NS
_CH = 128


def _sc_gather(table, ids):
    n = ids.shape[0]
    rank = table.shape[1]
    b_per_w = n // _NW
    n_ch = b_per_w // _CH
    ids3d = ids.reshape(_NW, n_ch, _CH)
    mesh = plsc.VectorSubcoreMesh(core_axis_name="c", subcore_axis_name="s")

    @functools.partial(
        pl.kernel,
        mesh=mesh,
        out_type=jax.ShapeDtypeStruct((n, rank), jnp.float32),
        scratch_types=[
            pltpu.VMEM((n_ch, _CH), jnp.int32),
            pltpu.VMEM((b_per_w, rank), jnp.float32),
            pltpu.SemaphoreType.DMA((n_ch,)),
            pltpu.SemaphoreType.DMA,
        ],
    )
    def gk(table_hbm, idx_hbm, out_hbm, idx_v, rows_v, gsems, wsem):
        wid = lax.axis_index("s") * _NC + lax.axis_index("c")
        base = wid * b_per_w
        pltpu.sync_copy(idx_hbm.at[wid], idx_v)
        copies = []
        for j in range(n_ch):
            copies.append(
                pltpu.async_copy(
                    table_hbm.at[idx_v.at[j]],
                    rows_v.at[pl.ds(j * _CH, _CH)],
                    gsems.at[j],
                )
            )
        writes = []
        for j in range(n_ch):
            copies[j].wait()
            writes.append(
                pltpu.async_copy(
                    rows_v.at[pl.ds(j * _CH, _CH)],
                    out_hbm.at[pl.ds(base + j * _CH, _CH)],
                    wsem,
                )
            )
        for w in writes:
            w.wait()

    return gk(table, ids3d)


def _mm_body(x_ref, w_ref, o_ref):
    o_ref[...] = lax.dot_general(
        x_ref[...].astype(jnp.bfloat16), w_ref[...].astype(jnp.bfloat16),
        dimension_numbers=(((1,), (1,)), ((), ())),
        preferred_element_type=jnp.float32,
    )


_BLK = 2048


def _tc_project(x, w):
    n_rows, rank = x.shape
    d_model = w.shape[0]
    return pl.pallas_call(
        _mm_body,
        grid=(n_rows // _BLK,),
        in_specs=[
            pl.BlockSpec((_BLK, rank), lambda i: (i, 0)),
            pl.BlockSpec((d_model, rank), lambda i: (0, 0)),
        ],
        out_specs=pl.BlockSpec((_BLK, d_model), lambda i: (i, 0)),
        out_shape=jax.ShapeDtypeStruct((n_rows, d_model), jnp.float32),
    )(x, w)


def _mm_into_body(prev_ref, x_ref, w_ref, o_ref):
    del prev_ref
    _mm_body(x_ref, w_ref, o_ref)


def _tc_project_into(prev, x, w, row0):
    n_rows, rank = x.shape
    d_model = w.shape[0]
    off = row0 // _BLK
    return pl.pallas_call(
        _mm_into_body,
        grid=(n_rows // _BLK,),
        in_specs=[
            pl.BlockSpec(memory_space=pl.MemorySpace.ANY),
            pl.BlockSpec((_BLK, rank), lambda i: (i, 0)),
            pl.BlockSpec((d_model, rank), lambda i: (0, 0)),
        ],
        out_specs=pl.BlockSpec((_BLK, d_model), lambda i, o=off: (i + o, 0)),
        out_shape=jax.ShapeDtypeStruct(prev.shape, jnp.float32),
        input_output_aliases={0: 0},
    )(prev, x, w)


def _tc_project_head(x, w, n_total):
    rank = x.shape[1]
    d_model = w.shape[0]
    return pl.pallas_call(
        _mm_body,
        grid=(x.shape[0] // _BLK,),
        in_specs=[
            pl.BlockSpec((_BLK, rank), lambda i: (i, 0)),
            pl.BlockSpec((d_model, rank), lambda i: (0, 0)),
        ],
        out_specs=pl.BlockSpec((_BLK, d_model), lambda i: (i, 0)),
        out_shape=jax.ShapeDtypeStruct((n_total, d_model), jnp.float32),
    )(x, w)


_SPLIT = 12288


def kernel(input_ids, embed_low, W_up):
    b, s = input_ids.shape
    n = b * s
    ids = input_ids.reshape(n).astype(jnp.int32)
    ids0 = lax.slice(ids, (0,), (_SPLIT,))
    ids1 = lax.slice(ids, (_SPLIT,), (n,))
    g0 = _sc_gather(embed_low, ids0)
    g1 = _sc_gather(embed_low, ids1)
    out = _tc_project_head(g0, W_up, n)
    out = _tc_project_into(out, g1, W_up, _SPLIT)
    return out.reshape(b, s, W_up.shape[0])


# speedup vs baseline: 1.0505x; 1.0505x over previous
import functools

import jax
import jax.numpy as jnp
from jax import lax
from jax.experimental import pallas as pl
from jax.experimental.pallas import tpu as pltpu
from jax.experimental.pallas import tpu_sc as plsc

_NC = 2
_NS = 16
_NW = _NC * _NS
_CH = 128


def _sc_gather(table, ids):
    n = ids.shape[0]
    rank = table.shape[1]
    b_per_w = n // _NW
    n_ch = b_per_w // _CH
    ids3d = ids.reshape(_NW, n_ch, _CH)
    mesh = plsc.VectorSubcoreMesh(core_axis_name="c", subcore_axis_name="s")

    @functools.partial(
        pl.kernel,
        mesh=mesh,
        out_type=jax.ShapeDtypeStruct((n, rank), jnp.float32),
        scratch_types=[
            pltpu.VMEM((n_ch, _CH), jnp.int32),
            pltpu.VMEM((b_per_w, rank), jnp.float32),
            pltpu.SemaphoreType.DMA((n_ch,)),
            pltpu.SemaphoreType.DMA,
        ],
    )
    def gk(table_hbm, idx_hbm, out_hbm, idx_v, rows_v, gsems, wsem):
        wid = lax.axis_index("s") * _NC + lax.axis_index("c")
        base = wid * b_per_w
        pltpu.sync_copy(idx_hbm.at[wid], idx_v)
        copies = []
        for j in range(n_ch):
            copies.append(
                pltpu.async_copy(
                    table_hbm.at[idx_v.at[j]],
                    rows_v.at[pl.ds(j * _CH, _CH)],
                    gsems.at[j],
                )
            )
        writes = []
        for j in range(n_ch):
            copies[j].wait()
            writes.append(
                pltpu.async_copy(
                    rows_v.at[pl.ds(j * _CH, _CH)],
                    out_hbm.at[pl.ds(base + j * _CH, _CH)],
                    wsem,
                )
            )
        for w in writes:
            w.wait()

    return gk(table, ids3d)


def _mm_body(x_ref, w_ref, o_ref):
    o_ref[...] = lax.dot_general(
        x_ref[...].astype(jnp.bfloat16), w_ref[...].astype(jnp.bfloat16),
        dimension_numbers=(((1,), (1,)), ((), ())),
        preferred_element_type=jnp.float32,
    )


_BLK = 2048


def _tc_project(x, w):
    n_rows, rank = x.shape
    d_model = w.shape[0]
    return pl.pallas_call(
        _mm_body,
        grid=(n_rows // _BLK,),
        in_specs=[
            pl.BlockSpec((_BLK, rank), lambda i: (i, 0)),
            pl.BlockSpec((d_model, rank), lambda i: (0, 0)),
        ],
        out_specs=pl.BlockSpec((_BLK, d_model), lambda i: (i, 0)),
        out_shape=jax.ShapeDtypeStruct((n_rows, d_model), jnp.float32),
    )(x, w)


def kernel(input_ids, embed_low, W_up):
    b, s = input_ids.shape
    n = b * s
    ids = input_ids.reshape(n).astype(jnp.int32)
    gathered = _sc_gather(embed_low, ids)
    out = _tc_project(gathered, W_up)
    return out.reshape(b, s, W_up.shape[0])
